# baseline (speedup 1.0000x reference)
import jax
import jax.numpy as jnp
from jax.experimental import pallas as pl
from jax.experimental.pallas import tpu as pltpu

H = 96
W = 96
N = H * W
NPAD = N + 256
CH = 64
BR = 256
CT = 512
NEG = -1e30


def _attn_kernel(pq_ref, f_blk, lq_ref, colneg_ref, out_blk):
    i = pl.program_id(0)
    p_cnt = pq_ref[0]
    q_cnt = pq_ref[1]

    @pl.when(i * BR < p_cnt)
    def _compute():
        f = f_blk[...]
        tj = (q_cnt + CT - 1) // CT

        def body(j, carry):
            m, s, acc = carry
            lt = lq_ref[pl.ds(j * CT, CT), :]
            inv = jax.lax.rsqrt(jnp.sum(lt * lt, axis=1, keepdims=True))
            ltn = lt * inv
            logits = jax.lax.dot_general(
                f, ltn, (((1,), (1,)), ((), ())),
                preferred_element_type=jnp.float32)
            logits = logits + colneg_ref[:, pl.ds(j * CT, CT)]
            m_new = jnp.maximum(m, jnp.max(logits, axis=1, keepdims=True))
            alpha = jnp.exp(m - m_new)
            p = jnp.exp(logits - m_new)
            s_new = s * alpha + jnp.sum(p, axis=1, keepdims=True)
            acc_new = acc * alpha + jax.lax.dot_general(
                p, lt, (((1,), (0,)), ((), ())),
                preferred_element_type=jnp.float32)
            return m_new, s_new, acc_new

        m0 = jnp.full((BR, 1), NEG, jnp.float32)
        s0 = jnp.zeros((BR, 1), jnp.float32)
        a0 = jnp.zeros((BR, CH), jnp.float32)
        m, s, acc = jax.lax.fori_loop(0, tj, body, (m0, s0, a0))
        out_blk[...] = acc / s

    @pl.when(i * BR >= p_cnt)
    def _zero():
        out_blk[...] = jnp.zeros((BR, CH), jnp.float32)


def kernel(input, mask, shift_sz, stride, triple_w, flag):
    bz, c, h, w = input.shape
    ch = c // 2
    f2d = input[0, :ch].reshape(ch, N).T
    l2d = input[0, ch:c].reshape(ch, N).T
    flag = flag.astype(jnp.int32)
    is_m = flag == 1
    iota = jnp.arange(N, dtype=jnp.int32)
    mcum = jnp.cumsum(is_m.astype(jnp.int32))
    p_cnt = mcum[-1]
    q_cnt = N - p_cnt

    rids = jnp.argsort(jnp.where(is_m, iota, iota + N))
    cids = jnp.argsort(jnp.where(is_m, iota + N, iota))
    fp = jnp.pad(jnp.take(f2d, rids, axis=0), ((0, NPAD - N), (0, 0)))
    lq = jnp.pad(jnp.take(l2d, cids, axis=0), ((0, NPAD - N), (0, 0)))
    colneg = jnp.where(jnp.arange(NPAD, dtype=jnp.int32) < q_cnt,
                       0.0, NEG).astype(jnp.float32).reshape(1, NPAD)
    pq = jnp.stack([p_cnt, q_cnt]).astype(jnp.int32)

    grid_spec = pltpu.PrefetchScalarGridSpec(
        num_scalar_prefetch=1,
        grid=(NPAD // BR,),
        in_specs=[
            pl.BlockSpec((BR, CH), lambda i, pq: (i, 0)),
            pl.BlockSpec((NPAD, CH), lambda i, pq: (0, 0)),
            pl.BlockSpec((1, NPAD), lambda i, pq: (0, 0)),
        ],
        out_specs=pl.BlockSpec((BR, CH), lambda i, pq: (i, 0)),
    )
    shifted = pl.pallas_call(
        _attn_kernel,
        grid_spec=grid_spec,
        out_shape=jax.ShapeDtypeStruct((NPAD, CH), jnp.float32),
        compiler_params=pltpu.CompilerParams(
            dimension_semantics=("arbitrary",)),
    )(pq, fp, lq, colneg)

    posx = jnp.where(is_m, mcum - 1, NPAD - 1)
    out2d = jnp.take(shifted, posx, axis=0)
    former_masked = out2d.T.reshape(1, ch, h, w)
    return jnp.concatenate([input, former_masked], axis=1)


# speedup vs baseline: 1.1244x; 1.1244x over previous
import functools

import jax
import jax.numpy as jnp
from jax.experimental import pallas as pl
from jax.experimental.pallas import tpu as pltpu
from jax.experimental.pallas import tpu_sc as plsc

H = 96
W = 96
N = H * W
NPAD = N + 256
CH = 64
CHP = 128
BR = 256
CT = 512
NEG = -1e30


def _attn_kernel(pq_ref, f_blk, lq_ref, colneg_ref, out_blk):
    i = pl.program_id(0)
    p_cnt = pq_ref[0]
    q_cnt = pq_ref[1]

    @pl.when(i * BR < p_cnt)
    def _compute():
        f = f_blk[...]
        tj = (q_cnt + CT - 1) // CT

        def body(j, carry):
            m, s, acc = carry
            lt = lq_ref[pl.ds(j * CT, CT), :]
            inv = jax.lax.rsqrt(jnp.sum(lt * lt, axis=1, keepdims=True))
            ltn = lt * inv
            logits = jax.lax.dot_general(
                f, ltn, (((1,), (1,)), ((), ())),
                preferred_element_type=jnp.float32)
            logits = logits + colneg_ref[:, pl.ds(j * CT, CT)]
            m_new = jnp.maximum(m, jnp.max(logits, axis=1, keepdims=True))
            alpha = jnp.exp(m - m_new)
            p = jnp.exp(logits - m_new)
            s_new = s * alpha + jnp.sum(p, axis=1, keepdims=True)
            acc_new = acc * alpha + jax.lax.dot_general(
                p, lt, (((1,), (0,)), ((), ())),
                preferred_element_type=jnp.float32)
            return m_new, s_new, acc_new

        m0 = jnp.full((BR, 1), NEG, jnp.float32)
        s0 = jnp.zeros((BR, 1), jnp.float32)
        a0 = jnp.zeros((BR, CHP), jnp.float32)
        m, s, acc = jax.lax.fori_loop(0, tj, body, (m0, s0, a0))
        out_blk[...] = acc / s

    @pl.when(i * BR >= p_cnt)
    def _zero():
        out_blk[...] = jnp.zeros((BR, CHP), jnp.float32)


NC = 2
NS = 16
NWORK = NC * NS
S = N // NWORK
JCH = 3
SC = S // JCH

@functools.cache
def _sc_mesh():
    return plsc.VectorSubcoreMesh(core_axis_name="c", subcore_axis_name="s")


def _compact_body(f2d, l2d, tgtf, tgtl, fp, lq, idxf_v, idxl_v, rows_f, rows_l, sem):
    wid = jax.lax.axis_index("s") * NC + jax.lax.axis_index("c")
    base = wid * S
    pltpu.sync_copy(tgtf.at[wid], idxf_v)
    pltpu.sync_copy(tgtl.at[wid], idxl_v)
    pltpu.sync_copy(f2d.at[pl.ds(base, S)], rows_f)
    pltpu.sync_copy(l2d.at[pl.ds(base, S)], rows_l)
    copies = []
    for j in range(JCH):
        copies.append(pltpu.async_copy(
            rows_f.at[pl.ds(j * SC, SC)], fp.at[idxf_v.at[j]], sem))
        copies.append(pltpu.async_copy(
            rows_l.at[pl.ds(j * SC, SC)], lq.at[idxl_v.at[j]], sem))
    for cp in copies:
        cp.wait()


def _sc_compact(f2d, l2d, tgtf, tgtl):
    return pl.kernel(
        _compact_body,
        out_type=(jax.ShapeDtypeStruct((NPAD, CHP), jnp.float32),
                  jax.ShapeDtypeStruct((NPAD, CHP), jnp.float32)),
        mesh=_sc_mesh(),
        scratch_types=[
            pltpu.VMEM((JCH, SC), jnp.int32),
            pltpu.VMEM((JCH, SC), jnp.int32),
            pltpu.VMEM((S, CHP), jnp.float32),
            pltpu.VMEM((S, CHP), jnp.float32),
            pltpu.SemaphoreType.DMA,
        ],
    )(f2d, l2d, tgtf, tgtl)


def _paste_body(shifted, posx, out, idx_v, rows_v, sem):
    wid = jax.lax.axis_index("s") * NC + jax.lax.axis_index("c")
    base = wid * S
    pltpu.sync_copy(posx.at[wid], idx_v)
    copies = [pltpu.async_copy(shifted.at[idx_v.at[j]],
                               rows_v.at[pl.ds(j * SC, SC)], sem)
              for j in range(JCH)]
    for cp in copies:
        cp.wait()
    pltpu.sync_copy(rows_v, out.at[pl.ds(base, S)])


def _sc_paste(shifted, posx):
    return pl.kernel(
        _paste_body,
        out_type=jax.ShapeDtypeStruct((N, CHP), jnp.float32),
        mesh=_sc_mesh(),
        scratch_types=[
            pltpu.VMEM((JCH, SC), jnp.int32),
            pltpu.VMEM((S, CHP), jnp.float32),
            pltpu.SemaphoreType.DMA,
        ],
    )(shifted, posx)


def kernel(input, mask, shift_sz, stride, triple_w, flag):
    bz, c, h, w = input.shape
    ch = c // 2
    f2d = jnp.pad(input[0, :ch].reshape(ch, N).T,
                  ((0, 0), (0, CHP - ch)))
    l2d = jnp.pad(input[0, ch:c].reshape(ch, N).T,
                  ((0, 0), (0, CHP - ch)))
    flag = flag.astype(jnp.int32)
    is_m = flag == 1
    mcum = jnp.cumsum(is_m.astype(jnp.int32))
    ucum = jnp.cumsum(1 - is_m.astype(jnp.int32))
    p_cnt = mcum[-1]
    q_cnt = N - p_cnt

    tgtf = jnp.where(is_m, mcum - 1, p_cnt + ucum - 1).reshape(NWORK, JCH, SC)
    tgtl = jnp.where(is_m, q_cnt + mcum - 1, ucum - 1).reshape(NWORK, JCH, SC)
    fp, lq = _sc_compact(f2d, l2d, tgtf, tgtl)
    colneg = jnp.where(jnp.arange(NPAD, dtype=jnp.int32) < q_cnt,
                       0.0, NEG).astype(jnp.float32).reshape(1, NPAD)
    pq = jnp.stack([p_cnt, q_cnt]).astype(jnp.int32)

    grid_spec = pltpu.PrefetchScalarGridSpec(
        num_scalar_prefetch=1,
        grid=(NPAD // BR,),
        in_specs=[
            pl.BlockSpec((BR, CHP), lambda i, pq: (i, 0)),
            pl.BlockSpec((NPAD, CHP), lambda i, pq: (0, 0)),
            pl.BlockSpec((1, NPAD), lambda i, pq: (0, 0)),
        ],
        out_specs=pl.BlockSpec((BR, CHP), lambda i, pq: (i, 0)),
    )
    shifted = pl.pallas_call(
        _attn_kernel,
        grid_spec=grid_spec,
        out_shape=jax.ShapeDtypeStruct((NPAD, CHP), jnp.float32),
        compiler_params=pltpu.CompilerParams(
            dimension_semantics=("arbitrary",)),
    )(pq, fp, lq, colneg)

    posx = jnp.where(is_m, mcum - 1, NPAD - 1).reshape(NWORK, JCH, SC)
    out2d = _sc_paste(shifted, posx)
    former_masked = out2d[:, :ch].T.reshape(1, ch, h, w)
    return jnp.concatenate([input, former_masked], axis=1)
